# baseline (speedup 1.0000x reference)
import functools

import jax
import jax.numpy as jnp
from jax import lax
from jax.experimental import pallas as pl
from jax.experimental.pallas import tpu as pltpu
from jax.experimental.pallas import tpu_sc as plsc

_V = 1000000
_D = 64
_B = 16384
_L = 50

_NC = 2
_NS = 16
_NW = _NC * _NS
_EPW = _B // _NW
_IPW = _EPW * 128
_EPR = 8
_RPW = _EPW // _EPR
_ROWS = _EPR * _L

_mesh = plsc.VectorSubcoreMesh(core_axis_name="c", subcore_axis_name="s")


@functools.partial(
    pl.kernel,
    mesh=_mesh,
    out_type=jax.ShapeDtypeStruct((_B, _D), jnp.float32),
    scratch_types=[
        pltpu.VMEM((_IPW,), jnp.int32),
        pltpu.VMEM((_ROWS, _D), jnp.float32),
        pltpu.VMEM((_ROWS, _D), jnp.float32),
        pltpu.VMEM((_EPR, _D), jnp.float32),
        pltpu.VMEM((_EPR, _D), jnp.float32),
        pltpu.SemaphoreType.DMA,
        pltpu.SemaphoreType.DMA,
        pltpu.SemaphoreType.DMA,
        pltpu.SemaphoreType.DMA,
    ],
    compiler_params=pltpu.CompilerParams(use_tc_tiling_on_sc=False),
)
def _gather_sums(x1_hbm, table_hbm, out_hbm, idx_v, buf0, buf1, acc0, acc1,
                 semg0, semg1, semo0, semo1):
    wid = lax.axis_index("s") * _NC + lax.axis_index("c")
    bufs = (buf0, buf1)
    accs = (acc0, acc1)
    semgs = (semg0, semg1)
    semos = (semo0, semo1)

    pltpu.sync_copy(
        x1_hbm.at[pl.ds(pl.multiple_of(wid * _IPW, 8), _IPW)], idx_v
    )

    def fire(r, slot):
        for ee in range(_EPR):
            w = pl.multiple_of((r * _EPR + ee) * 128, 8)
            pltpu.async_copy(
                table_hbm.at[idx_v.at[pl.ds(w, _L)]],
                bufs[slot].at[pl.ds(ee * _L, _L), :],
                semgs[slot],
            )

    def drain_gathers(slot):
        for ee in range(_EPR):
            pltpu.make_async_copy(
                table_hbm.at[idx_v.at[pl.ds(0, _L)]],
                bufs[slot].at[pl.ds(ee * _L, _L), :],
                semgs[slot],
            ).wait()

    def out_start(r, slot):
        pltpu.async_copy(
            accs[slot],
            out_hbm.at[pl.ds(wid * _EPW + r * _EPR, _EPR), :],
            semos[slot],
        )

    def out_wait(slot):
        pltpu.make_async_copy(
            accs[slot],
            out_hbm.at[pl.ds(wid * _EPW, _EPR), :],
            semos[slot],
        ).wait()

    def accumulate(slot):
        buf, acc = bufs[slot], accs[slot]
        for e in range(_EPR):
            base = e * _L
            a0 = tuple(buf[base, pl.ds(16 * v, 16)] for v in range(4))

            def body7(k, a, base=base, buf=buf):
                row = base + 1 + k * 7
                for u in range(7):
                    a = tuple(
                        a[v] + buf[row + u, pl.ds(16 * v, 16)]
                        for v in range(4)
                    )
                return a

            a = lax.fori_loop(0, 7, body7, a0)
            for v in range(4):
                acc[e, pl.ds(16 * v, 16)] = a[v]

    fire(0, 0)

    def round_pair(j, carry):
        fire(2 * j + 1, 1)
        drain_gathers(0)

        @pl.when(j > 0)
        def _():
            out_wait(0)

        accumulate(0)
        out_start(2 * j, 0)

        @pl.when(j < _RPW // 2 - 1)
        def _():
            fire(2 * j + 2, 0)

        drain_gathers(1)

        @pl.when(j > 0)
        def _():
            out_wait(1)

        accumulate(1)
        out_start(2 * j + 1, 1)
        return carry

    lax.fori_loop(0, _RPW // 2, round_pair, 0)
    out_wait(0)
    out_wait(1)


def _tc_body(h_ref, w_ref, b_ref, g_ref, beta_ref, o_ref):
    y = (
        jnp.dot(h_ref[...], w_ref[...], preferred_element_type=jnp.float32)
        + b_ref[...]
    )
    mu = jnp.mean(y, axis=-1, keepdims=True)
    yc = y - mu
    var = jnp.mean(yc * yc, axis=-1, keepdims=True)
    o_ref[...] = yc * lax.rsqrt(var + 1e-3) * g_ref[...] + beta_ref[...]


_TB = 1024


def _pad_body(x_ref, o_ref):
    o_ref[...] = jnp.pad(x_ref[...], ((0, 0), (0, 128 - _L)))


_RB = 2048


def kernel(x, table, W, b, bn_gamma, bn_beta, bn_mean, bn_var, ln_gamma, ln_beta):
    x128 = pl.pallas_call(
        _pad_body,
        grid=(_B // _RB,),
        in_specs=[pl.BlockSpec((_RB, _L), lambda i: (i, 0))],
        out_specs=pl.BlockSpec((_RB, 128), lambda i: (i, 0)),
        out_shape=jax.ShapeDtypeStruct((_B, 128), jnp.int32),
    )(x.astype(jnp.int32))
    x1 = x128.reshape(-1)

    sums = _gather_sums(x1, table)

    s = bn_gamma * lax.rsqrt(bn_var + 1e-3)
    Wp = W * (s[None, :] / _L)
    bp = ((b - bn_mean) * s + bn_beta).reshape(1, _D)

    return pl.pallas_call(
        _tc_body,
        grid=(_B // _TB,),
        in_specs=[
            pl.BlockSpec((_TB, _D), lambda i: (i, 0)),
            pl.BlockSpec((_D, _D), lambda i: (0, 0)),
            pl.BlockSpec((1, _D), lambda i: (0, 0)),
            pl.BlockSpec((1, _D), lambda i: (0, 0)),
            pl.BlockSpec((1, _D), lambda i: (0, 0)),
        ],
        out_specs=pl.BlockSpec((_TB, _D), lambda i: (i, 0)),
        out_shape=jax.ShapeDtypeStruct((_B, _D), jnp.float32),
    )(sums, Wp, bp, ln_gamma.reshape(1, _D), ln_beta.reshape(1, _D))


# speedup vs baseline: 1.0107x; 1.0107x over previous
import functools

import jax
import jax.numpy as jnp
from jax import lax
from jax.experimental import pallas as pl
from jax.experimental.pallas import tpu as pltpu
from jax.experimental.pallas import tpu_sc as plsc

_V = 1000000
_D = 64
_B = 16384
_L = 50

_NC = 2
_NS = 16
_NW = _NC * _NS
_EPW = _B // _NW
_IPW = _EPW * 128
_EPR = 8
_RPW = _EPW // _EPR
_ROWS = _EPR * _L

_mesh = plsc.VectorSubcoreMesh(core_axis_name="c", subcore_axis_name="s")


@functools.partial(
    pl.kernel,
    mesh=_mesh,
    out_type=jax.ShapeDtypeStruct((_B, 128), jnp.float32),
    scratch_types=[
        pltpu.VMEM((_IPW,), jnp.int32),
        pltpu.VMEM((_ROWS, _D), jnp.float32),
        pltpu.VMEM((_ROWS, _D), jnp.float32),
        pltpu.VMEM((_EPR, 128), jnp.float32),
        pltpu.VMEM((_EPR, 128), jnp.float32),
        pltpu.SemaphoreType.DMA,
        pltpu.SemaphoreType.DMA,
        pltpu.SemaphoreType.DMA,
        pltpu.SemaphoreType.DMA,
    ],
    compiler_params=pltpu.CompilerParams(use_tc_tiling_on_sc=False),
)
def _gather_sums(x1_hbm, table_hbm, out_hbm, idx_v, buf0, buf1, acc0, acc1,
                 semg0, semg1, semo0, semo1):
    wid = lax.axis_index("s") * _NC + lax.axis_index("c")
    bufs = (buf0, buf1)
    accs = (acc0, acc1)
    semgs = (semg0, semg1)
    semos = (semo0, semo1)

    pltpu.sync_copy(
        x1_hbm.at[pl.ds(pl.multiple_of(wid * _IPW, 8), _IPW)], idx_v
    )

    def fire(r, slot):
        for ee in range(_EPR):
            w = pl.multiple_of((r * _EPR + ee) * 128, 8)
            pltpu.async_copy(
                table_hbm.at[idx_v.at[pl.ds(w, _L)]],
                bufs[slot].at[pl.ds(ee * _L, _L), :],
                semgs[slot],
            )

    def drain_gathers(slot):
        for ee in range(_EPR):
            pltpu.make_async_copy(
                table_hbm.at[idx_v.at[pl.ds(0, _L)]],
                bufs[slot].at[pl.ds(ee * _L, _L), :],
                semgs[slot],
            ).wait()

    def out_start(r, slot):
        pltpu.async_copy(
            accs[slot],
            out_hbm.at[pl.ds(wid * _EPW + r * _EPR, _EPR), :],
            semos[slot],
        )

    def out_wait(slot):
        pltpu.make_async_copy(
            accs[slot],
            out_hbm.at[pl.ds(wid * _EPW, _EPR), :],
            semos[slot],
        ).wait()

    def accumulate(slot):
        buf, acc = bufs[slot], accs[slot]
        for e in range(_EPR):
            base = e * _L
            a0 = tuple(buf[base, pl.ds(16 * v, 16)] for v in range(4))

            def body7(k, a, base=base, buf=buf):
                row = base + 1 + k * 7
                for u in range(7):
                    a = tuple(
                        a[v] + buf[row + u, pl.ds(16 * v, 16)]
                        for v in range(4)
                    )
                return a

            a = lax.fori_loop(0, 7, body7, a0)
            for v in range(4):
                acc[e, pl.ds(16 * v, 16)] = a[v]

    fire(0, 0)

    def round_pair(j, carry):
        fire(2 * j + 1, 1)
        drain_gathers(0)

        @pl.when(j > 0)
        def _():
            out_wait(0)

        accumulate(0)
        out_start(2 * j, 0)

        @pl.when(j < _RPW // 2 - 1)
        def _():
            fire(2 * j + 2, 0)

        drain_gathers(1)

        @pl.when(j > 0)
        def _():
            out_wait(1)

        accumulate(1)
        out_start(2 * j + 1, 1)
        return carry

    lax.fori_loop(0, _RPW // 2, round_pair, 0)
    out_wait(0)
    out_wait(1)


def _tc_body(h_ref, w_ref, b_ref, g_ref, beta_ref, o_ref):
    h = h_ref[:, : _D]
    y = (
        jnp.dot(h, w_ref[...], preferred_element_type=jnp.float32)
        + b_ref[...]
    )
    mu = jnp.mean(y, axis=-1, keepdims=True)
    yc = y - mu
    var = jnp.mean(yc * yc, axis=-1, keepdims=True)
    o_ref[...] = yc * lax.rsqrt(var + 1e-3) * g_ref[...] + beta_ref[...]


_TB = 1024


def _pad_body(x_ref, o_ref):
    o_ref[...] = jnp.pad(x_ref[...], ((0, 0), (0, 128 - _L)))


_RB = 2048


def kernel(x, table, W, b, bn_gamma, bn_beta, bn_mean, bn_var, ln_gamma, ln_beta):
    x128 = pl.pallas_call(
        _pad_body,
        grid=(_B // _RB,),
        in_specs=[pl.BlockSpec((_RB, _L), lambda i: (i, 0))],
        out_specs=pl.BlockSpec((_RB, 128), lambda i: (i, 0)),
        out_shape=jax.ShapeDtypeStruct((_B, 128), jnp.int32),
    )(x.astype(jnp.int32))
    x1 = x128.reshape(-1)

    sums = _gather_sums(x1, table)

    s = bn_gamma * lax.rsqrt(bn_var + 1e-3)
    Wp = W * (s[None, :] / _L)
    bp = ((b - bn_mean) * s + bn_beta).reshape(1, _D)

    return pl.pallas_call(
        _tc_body,
        grid=(_B // _TB,),
        in_specs=[
            pl.BlockSpec((_TB, 128), lambda i: (i, 0)),
            pl.BlockSpec((_D, _D), lambda i: (0, 0)),
            pl.BlockSpec((1, _D), lambda i: (0, 0)),
            pl.BlockSpec((1, _D), lambda i: (0, 0)),
            pl.BlockSpec((1, _D), lambda i: (0, 0)),
        ],
        out_specs=pl.BlockSpec((_TB, _D), lambda i: (i, 0)),
        out_shape=jax.ShapeDtypeStruct((_B, _D), jnp.float32),
    )(sums, Wp, bp, ln_gamma.reshape(1, _D), ln_beta.reshape(1, _D))
